# baseline (speedup 1.0000x reference)
import functools

import jax
import jax.numpy as jnp
from jax import lax
from jax.experimental import pallas as pl
from jax.experimental.pallas import tpu as pltpu
from jax.experimental.pallas import tpu_sc as plsc

V = 100000
R = 7
M = 6
NW = 32
PV = 100096
CV = 128
IC = CV * R * M
NI = V * R * M
NCHUNKS = PV // CV
NITER = -(-NCHUNKS // NW)


def _sc_gather_mean(x, idx):
    mesh = plsc.VectorSubcoreMesh(core_axis_name="c", subcore_axis_name="s")

    @functools.partial(
        pl.kernel,
        out_type=jax.ShapeDtypeStruct((R, PV), jnp.float32),
        mesh=mesh,
        compiler_params=pltpu.CompilerParams(needs_layout_passes=False),
        scratch_types=[
            pltpu.VMEM((V,), jnp.float32),
            pltpu.VMEM((IC,), jnp.int32),
            pltpu.VMEM((IC,), jnp.int32),
            pltpu.VMEM((R, CV), jnp.float32),
            pltpu.VMEM((R, CV), jnp.float32),
            pltpu.SemaphoreType.DMA,
            pltpu.SemaphoreType.DMA,
            pltpu.SemaphoreType.DMA,
            pltpu.SemaphoreType.DMA,
            pltpu.SemaphoreType.DMA,
        ],
    )
    def k(x_hbm, idx_hbm, out_hbm, table_v, idx_v0, idx_v1, out_v0, out_v1,
          sem_t, sem_i0, sem_i1, sem_o0, sem_o1):
        wid = lax.axis_index("s") * 2 + lax.axis_index("c")
        idx_bufs = (idx_v0, idx_v1)
        out_bufs = (out_v0, out_v1)
        idx_sems = (sem_i0, sem_i1)
        out_sems = (sem_o0, sem_o1)
        iota42 = lax.iota(jnp.int32, 16) * (R * M)
        njlast = (V - (NCHUNKS - 1) * CV) // 16

        def cid(g):
            return jnp.minimum(wid + NW * g, NCHUNKS - 1)

        def ibase(c):
            return jnp.minimum(c * IC, NI - IC)

        def fire_idx(g, b):
            return pltpu.async_copy(
                idx_hbm.at[pl.ds(ibase(cid(g)), IC)], idx_bufs[b],
                idx_sems[b])

        def wait_idx(g, b):
            pltpu.make_async_copy(
                idx_hbm.at[pl.ds(ibase(cid(g)), IC)], idx_bufs[b],
                idx_sems[b]).wait()

        def fire_out(g, b):
            pltpu.async_copy(
                out_bufs[b], out_hbm.at[:, pl.ds(cid(g) * CV, CV)],
                out_sems[b])

        def wait_out(g, b):
            pltpu.make_async_copy(
                out_bufs[b], out_hbm.at[:, pl.ds(cid(g) * CV, CV)],
                out_sems[b]).wait()

        def compute(g, idx_ref, out_ref):
            c = cid(g)
            shift = c * IC - ibase(c)
            nj = jnp.where(jnp.equal(c, NCHUNKS - 1), njlast, CV // 16)
            for r in range(R):
                @plsc.parallel_loop(0, nj, unroll=2)
                def _(j):
                    u0 = j * 16
                    acc = jnp.zeros((16,), jnp.float32)
                    for m in range(M):
                        sv = iota42 + (shift + u0 * (R * M) + r * M + m)
                        iv = plsc.load_gather(idx_ref, [sv])
                        acc = acc + plsc.load_gather(table_v, [iv])
                    out_ref[r, pl.ds(u0, 16)] = acc * (1.0 / M)

        t_copy = pltpu.async_copy(x_hbm.at[0], table_v, sem_t)
        fire_idx(0, 0)
        fire_idx(1, 1)
        t_copy.wait()

        for b in range(2):
            wait_idx(b, b)
            compute(b, idx_bufs[b], out_bufs[b])
            fire_out(b, b)
            fire_idx(b + 2, b)

        @pl.loop(1, (NITER - 1) // 2)
        def _(t):
            for b in range(2):
                g = 2 * t + b
                wait_idx(g, b)
                wait_out(g - 2, b)
                compute(g, idx_bufs[b], out_bufs[b])
                fire_out(g, b)

                @pl.when(g + 2 < NITER)
                def _():
                    fire_idx(g + 2, b)

        g_last = NITER - 1
        wait_idx(g_last, 0)
        wait_out(g_last - 2, 0)
        compute(g_last, idx_bufs[0], out_bufs[0])
        fire_out(g_last, 0)

        wait_out(g_last - 1, 1)
        wait_out(g_last, 0)

    return k(x, idx)


def kernel(x, indices):
    idx = indices.astype(jnp.int32)
    out = _sc_gather_mean(x, idx)
    out = out[:, :V]
    return jnp.reshape(jnp.transpose(out, (1, 0)), (1, V, R))


# speedup vs baseline: 1.1170x; 1.1170x over previous
import functools

import jax
import jax.numpy as jnp
from jax import lax
from jax.experimental import pallas as pl
from jax.experimental.pallas import tpu as pltpu
from jax.experimental.pallas import tpu_sc as plsc

V = 100000
R = 7
M = 6
NW = 32
PV = 100096
CV = 128
IC = CV * R * M
NI = V * R * M
NCHUNKS = PV // CV
NITER = -(-NCHUNKS // NW)


def _sc_gather_mean(x, idx):
    mesh = plsc.VectorSubcoreMesh(core_axis_name="c", subcore_axis_name="s")

    @functools.partial(
        pl.kernel,
        out_type=jax.ShapeDtypeStruct((R, PV), jnp.float32),
        mesh=mesh,
        compiler_params=pltpu.CompilerParams(needs_layout_passes=False),
        scratch_types=[
            pltpu.VMEM((V,), jnp.float32),
            pltpu.VMEM_SHARED((V,), jnp.float32),
            pltpu.VMEM((IC,), jnp.int32),
            pltpu.VMEM((IC,), jnp.int32),
            pltpu.VMEM((R, CV), jnp.float32),
            pltpu.VMEM((R, CV), jnp.float32),
            pltpu.SemaphoreType.DMA,
            pltpu.SemaphoreType.DMA,
            pltpu.SemaphoreType.DMA,
            pltpu.SemaphoreType.DMA,
            pltpu.SemaphoreType.DMA,
        ],
    )
    def k(x_hbm, idx_hbm, out_hbm, table_v, table_s, idx_v0, idx_v1,
          out_v0, out_v1, sem_t, sem_i0, sem_i1, sem_o0, sem_o1):
        wid = lax.axis_index("s") * 2 + lax.axis_index("c")
        idx_bufs = (idx_v0, idx_v1)
        out_bufs = (out_v0, out_v1)
        idx_sems = (sem_i0, sem_i1)
        out_sems = (sem_o0, sem_o1)
        iota42 = lax.iota(jnp.int32, 16) * (R * M)
        njlast = (V - (NCHUNKS - 1) * CV) // 16

        def cid(g):
            return jnp.minimum(wid + NW * g, NCHUNKS - 1)

        def ibase(c):
            return jnp.minimum(c * IC, NI - IC)

        def fire_idx(g, b):
            return pltpu.async_copy(
                idx_hbm.at[pl.ds(ibase(cid(g)), IC)], idx_bufs[b],
                idx_sems[b])

        def wait_idx(g, b):
            pltpu.make_async_copy(
                idx_hbm.at[pl.ds(ibase(cid(g)), IC)], idx_bufs[b],
                idx_sems[b]).wait()

        def fire_out(g, b):
            pltpu.async_copy(
                out_bufs[b], out_hbm.at[:, pl.ds(cid(g) * CV, CV)],
                out_sems[b])

        def wait_out(g, b):
            pltpu.make_async_copy(
                out_bufs[b], out_hbm.at[:, pl.ds(cid(g) * CV, CV)],
                out_sems[b]).wait()

        def compute(g, idx_ref, out_ref):
            c = cid(g)
            shift = c * IC - ibase(c)
            nj = jnp.where(jnp.equal(c, NCHUNKS - 1), njlast, CV // 16)
            for r in range(R):
                @plsc.parallel_loop(0, nj, unroll=2)
                def _(j):
                    u0 = j * 16
                    acc = jnp.zeros((16,), jnp.float32)
                    for m in range(M):
                        sv = iota42 + (shift + u0 * (R * M) + r * M + m)
                        iv = plsc.load_gather(idx_ref, [sv])
                        acc = acc + plsc.load_gather(table_v, [iv])
                    out_ref[r, pl.ds(u0, 16)] = acc * (1.0 / M)

        sid = lax.axis_index("s")
        fire_idx(0, 0)
        fire_idx(1, 1)

        @pl.when(jnp.equal(sid, 0))
        def _():
            pltpu.async_copy(x_hbm.at[0], table_s, sem_t).wait()

        plsc.subcore_barrier()
        pltpu.sync_copy(table_s, table_v)

        for b in range(2):
            wait_idx(b, b)
            compute(b, idx_bufs[b], out_bufs[b])
            fire_out(b, b)
            fire_idx(b + 2, b)

        @pl.loop(1, (NITER - 1) // 2)
        def _(t):
            for b in range(2):
                g = 2 * t + b
                wait_idx(g, b)
                wait_out(g - 2, b)
                compute(g, idx_bufs[b], out_bufs[b])
                fire_out(g, b)

                @pl.when(g + 2 < NITER)
                def _():
                    fire_idx(g + 2, b)

        g_last = NITER - 1
        wait_idx(g_last, 0)
        wait_out(g_last - 2, 0)
        compute(g_last, idx_bufs[0], out_bufs[0])
        fire_out(g_last, 0)

        wait_out(g_last - 1, 1)
        wait_out(g_last, 0)

    return k(x, idx)


def kernel(x, indices):
    idx = indices.astype(jnp.int32)
    out = _sc_gather_mean(x, idx)
    out = out[:, :V]
    return jnp.reshape(jnp.transpose(out, (1, 0)), (1, V, R))


# speedup vs baseline: 1.2030x; 1.0770x over previous
import functools

import jax
import jax.numpy as jnp
from jax import lax
from jax.experimental import pallas as pl
from jax.experimental.pallas import tpu as pltpu
from jax.experimental.pallas import tpu_sc as plsc

V = 100000
R = 7
M = 6
NW = 32
PV = 100096
CV = 128
IC = CV * R * M
NI = V * R * M
NCHUNKS = PV // CV
NITER = -(-NCHUNKS // NW)


def _sc_gather_mean(x, idx):
    mesh = plsc.VectorSubcoreMesh(core_axis_name="c", subcore_axis_name="s")

    @functools.partial(
        pl.kernel,
        out_type=jax.ShapeDtypeStruct((R, PV), jnp.float32),
        mesh=mesh,
        compiler_params=pltpu.CompilerParams(needs_layout_passes=False),
        scratch_types=[
            pltpu.VMEM((V,), jnp.float32),
            pltpu.VMEM_SHARED((V,), jnp.float32),
            pltpu.VMEM((IC,), jnp.int32),
            pltpu.VMEM((IC,), jnp.int32),
            pltpu.VMEM((R, CV), jnp.float32),
            pltpu.VMEM((R, CV), jnp.float32),
            pltpu.SemaphoreType.DMA,
            pltpu.SemaphoreType.DMA,
            pltpu.SemaphoreType.DMA,
            pltpu.SemaphoreType.DMA,
            pltpu.SemaphoreType.DMA,
        ],
    )
    def k(x_hbm, idx_hbm, out_hbm, table_v, table_s, idx_v0, idx_v1,
          out_v0, out_v1, sem_t, sem_i0, sem_i1, sem_o0, sem_o1):
        wid = lax.axis_index("s") * 2 + lax.axis_index("c")
        idx_bufs = (idx_v0, idx_v1)
        out_bufs = (out_v0, out_v1)
        idx_sems = (sem_i0, sem_i1)
        out_sems = (sem_o0, sem_o1)
        iota42 = lax.iota(jnp.int32, 16) * (R * M)
        njlast = (V - (NCHUNKS - 1) * CV) // 16

        def cid(g):
            return jnp.minimum(wid + NW * g, NCHUNKS - 1)

        def ibase(c):
            return jnp.minimum(c * IC, NI - IC)

        def fire_idx(g, b):
            return pltpu.async_copy(
                idx_hbm.at[pl.ds(ibase(cid(g)), IC)], idx_bufs[b],
                idx_sems[b])

        def wait_idx(g, b):
            pltpu.make_async_copy(
                idx_hbm.at[pl.ds(ibase(cid(g)), IC)], idx_bufs[b],
                idx_sems[b]).wait()

        def fire_out(g, b):
            pltpu.async_copy(
                out_bufs[b], out_hbm.at[:, pl.ds(cid(g) * CV, CV)],
                out_sems[b])

        def wait_out(g, b):
            pltpu.make_async_copy(
                out_bufs[b], out_hbm.at[:, pl.ds(cid(g) * CV, CV)],
                out_sems[b]).wait()

        def compute(g, idx_ref, out_ref):
            c = cid(g)
            shift = c * IC - ibase(c)
            nj = jnp.where(jnp.equal(c, NCHUNKS - 1), njlast, CV // 16)
            for r in range(R):
                @plsc.parallel_loop(0, nj, unroll=2)
                def _(j):
                    u0 = j * 16
                    acc = jnp.zeros((16,), jnp.float32)
                    for m in range(M):
                        sv = iota42 + (shift + u0 * (R * M) + r * M + m)
                        iv = plsc.load_gather(idx_ref, [sv])
                        acc = acc + plsc.load_gather(table_v, [iv])
                    out_ref[r, pl.ds(u0, 16)] = acc * (1.0 / M)

        sid = lax.axis_index("s")
        fire_idx(0, 0)
        fire_idx(1, 1)

        @pl.when(jnp.equal(sid, 0))
        def _():
            pltpu.async_copy(x_hbm.at[0], table_s, sem_t).wait()

        plsc.subcore_barrier()
        pltpu.sync_copy(table_s, table_v)

        @pl.loop(0, (NITER + 1) // 2)
        def _(t):
            for b in range(2):
                g = 2 * t + b

                @pl.when(g < NITER)
                def _():
                    wait_idx(g, b)

                    @pl.when(g >= 2)
                    def _():
                        wait_out(g - 2, b)

                    compute(g, idx_bufs[b], out_bufs[b])
                    fire_out(g, b)

                    @pl.when(g + 2 < NITER)
                    def _():
                        fire_idx(g + 2, b)

        wait_out(NITER - 2, 1)
        wait_out(NITER - 1, 0)

    return k(x, idx)


def kernel(x, indices):
    idx = indices.astype(jnp.int32)
    out = _sc_gather_mean(x, idx)
    out = out[:, :V]
    return jnp.reshape(jnp.transpose(out, (1, 0)), (1, V, R))
